# baseline (speedup 1.0000x reference)
import functools

import jax
import jax.numpy as jnp
from jax import lax
from jax.experimental import pallas as pl
from jax.experimental.pallas import tpu as pltpu
from jax.experimental.pallas import tpu_sc as plsc

_CHUNK = 128


@functools.cache
def _make_gather(V: int, D: int, B: int):
  info = plsc.get_sparse_core_info()
  nw = info.num_cores * info.num_subcores
  b_per_w = B // nw
  n_chunks = b_per_w // _CHUNK
  mesh = plsc.VectorSubcoreMesh(core_axis_name="c", subcore_axis_name="s")

  @functools.partial(
      pl.kernel,
      mesh=mesh,
      out_type=jax.ShapeDtypeStruct((B, D), jnp.float32),
      scratch_types=[
          pltpu.VMEM((n_chunks, _CHUNK), jnp.int32),
          pltpu.VMEM((b_per_w, D), jnp.float32),
          pltpu.SemaphoreType.DMA,
      ],
      compiler_params=pltpu.CompilerParams(use_tc_tiling_on_sc=False),
  )
  def gather_kernel(idx_hbm, table_hbm, out_hbm, idx_v, rows_v, sem):
    wid = lax.axis_index("s") * info.num_cores + lax.axis_index("c")
    base = wid * b_per_w
    pltpu.sync_copy(idx_hbm.at[pl.ds(wid * n_chunks, n_chunks)], idx_v)
    copies = [
        pltpu.async_copy(
            table_hbm.at[idx_v.at[j]],
            rows_v.at[pl.ds(j * _CHUNK, _CHUNK)],
            sem,
        )
        for j in range(n_chunks)
    ]
    for c in copies:
      c.wait()
    pltpu.sync_copy(rows_v, out_hbm.at[pl.ds(base, b_per_w)])

  return gather_kernel


def kernel(input, action_embedding):
  B = input.shape[0]
  V, D = action_embedding.shape
  idx2d = input.astype(jnp.int32).reshape(B // _CHUNK, _CHUNK)
  return _make_gather(V, D, B)(idx2d, action_embedding)


# speedup vs baseline: 1.0001x; 1.0001x over previous
import functools

import jax
import jax.numpy as jnp
from jax import lax
from jax.experimental import pallas as pl
from jax.experimental.pallas import tpu as pltpu
from jax.experimental.pallas import tpu_sc as plsc

_CHUNK = 128


@functools.cache
def _make_gather(V: int, D: int, B: int):
  info = plsc.get_sparse_core_info()
  nw = info.num_cores * info.num_subcores
  b_per_w = B // nw
  n_chunks = b_per_w // _CHUNK
  mesh = plsc.VectorSubcoreMesh(core_axis_name="c", subcore_axis_name="s")

  @functools.partial(
      pl.kernel,
      mesh=mesh,
      out_type=jax.ShapeDtypeStruct((B, D), jnp.float32),
      scratch_types=[
          pltpu.VMEM((n_chunks, _CHUNK), jnp.int32),
          pltpu.VMEM((b_per_w, D), jnp.float32),
          pltpu.SemaphoreType.DMA((n_chunks,)),
          pltpu.SemaphoreType.DMA,
      ],
      compiler_params=pltpu.CompilerParams(use_tc_tiling_on_sc=False),
  )
  def gather_kernel(idx_hbm, table_hbm, out_hbm, idx_v, rows_v, gsem, wsem):
    wid = lax.axis_index("s") * info.num_cores + lax.axis_index("c")
    base = wid * b_per_w
    pltpu.sync_copy(idx_hbm.at[pl.ds(wid * n_chunks, n_chunks)], idx_v)
    gathers = [
        pltpu.async_copy(
            table_hbm.at[idx_v.at[j]],
            rows_v.at[pl.ds(j * _CHUNK, _CHUNK)],
            gsem.at[j],
        )
        for j in range(n_chunks)
    ]
    writes = []
    for j in range(n_chunks):
      gathers[j].wait()
      writes.append(
          pltpu.async_copy(
              rows_v.at[pl.ds(j * _CHUNK, _CHUNK)],
              out_hbm.at[pl.ds(base + j * _CHUNK, _CHUNK)],
              wsem,
          )
      )
    for w in writes:
      w.wait()

  return gather_kernel


def kernel(input, action_embedding):
  B = input.shape[0]
  V, D = action_embedding.shape
  idx2d = input.astype(jnp.int32).reshape(B // _CHUNK, _CHUNK)
  return _make_gather(V, D, B)(idx2d, action_embedding)


# speedup vs baseline: 3.0444x; 3.0440x over previous
import functools

import jax
import jax.numpy as jnp
from jax import lax
from jax.experimental import pallas as pl
from jax.experimental.pallas import tpu as pltpu
from jax.experimental.pallas import tpu_sc as plsc

_RING = 8


@functools.cache
def _make_gather(V: int, D: int, B: int):
  info = plsc.get_sparse_core_info()
  nw = info.num_cores * info.num_subcores
  b_per_w = B // nw
  n_groups = b_per_w // _RING
  mesh = plsc.VectorSubcoreMesh(core_axis_name="c", subcore_axis_name="s")
  tail_start = V - 128
  tail_cut = (V // 128) * 128

  @functools.partial(
      pl.kernel,
      mesh=mesh,
      out_type=jax.ShapeDtypeStruct((D, B), jnp.float32),
      scratch_types=[
          pltpu.VMEM((b_per_w + 16,), jnp.int32),
          [pltpu.VMEM((D, 128), jnp.float32) for _ in range(_RING)],
          pltpu.VMEM((D, b_per_w), jnp.float32),
          pltpu.SemaphoreType.DMA((_RING,)),
      ],
      compiler_params=pltpu.CompilerParams(needs_layout_passes=False),
  )
  def gather_kernel(idx_hbm, table_hbm, aux_hbm, out_hbm, idx_v, slabs, out_v,
                    sems):
    wid = lax.axis_index("s") * info.num_cores + lax.axis_index("c")
    base = wid * b_per_w
    pltpu.sync_copy(idx_hbm.at[pl.ds(base, b_per_w)], idx_v.at[pl.ds(0, b_per_w)])

    rows = [lax.iota(jnp.int32, 16) + 16 * k for k in range(4)]

    def fire(slot, a):
      c = pl.multiple_of((jnp.minimum(a, tail_cut - 1) // 128) * 128, 128)

      @pl.when(a < tail_cut)
      def _():
        pltpu.async_copy(table_hbm.at[:, pl.ds(c, 128)], slabs[slot],
                         sems.at[slot])

      @pl.when(a >= tail_cut)
      def _():
        pltpu.async_copy(aux_hbm, slabs[slot], sems.at[slot])

    vec0 = idx_v[pl.ds(0, 16)]
    for j in range(_RING):
      fire(j, vec0[j])

    def group(i):
      cur = idx_v[pl.ds(i * _RING, 16)]
      nxt = idx_v[pl.ds(i * _RING + _RING, 16)]
      for j in range(_RING):
        a = cur[j]
        l = jnp.where(a >= tail_cut, a - tail_start, a % 128)
        lane = jnp.full((16,), l, jnp.int32)
        pos = jnp.full((16,), i * _RING + j, jnp.int32)
        pltpu.make_async_copy(
            table_hbm.at[:, pl.ds(0, 128)], slabs[j], sems.at[j]
        ).wait()
        for k in range(4):
          vals = plsc.load_gather(slabs[j], [rows[k], lane])
          plsc.store_scatter(out_v, [rows[k], pos], vals)

        @pl.when(i < n_groups - 1)
        def _():
          fire(j, nxt[j])

    pl.loop(0, n_groups)(group)
    pltpu.sync_copy(out_v, out_hbm.at[:, pl.ds(base, b_per_w)])

  return gather_kernel


def kernel(input, action_embedding):
  B = input.shape[0]
  V, D = action_embedding.shape
  idx = input.astype(jnp.int32).reshape(B)
  aux = action_embedding[V - 128:].T
  out_t = _make_gather(V, D, B)(idx, action_embedding.T, aux)
  return out_t.T


# speedup vs baseline: 3.0520x; 1.0025x over previous
import functools

import jax
import jax.numpy as jnp
from jax import lax
from jax.experimental import pallas as pl
from jax.experimental.pallas import tpu as pltpu
from jax.experimental.pallas import tpu_sc as plsc

_RING = 8


@functools.cache
def _make_gather(V: int, D: int, B: int):
  info = plsc.get_sparse_core_info()
  nw = info.num_cores * info.num_subcores
  b_per_w = B // nw
  n_groups = b_per_w // _RING
  mesh = plsc.VectorSubcoreMesh(core_axis_name="c", subcore_axis_name="s")
  tail_start = V - 128
  tail_cut = (V // 128) * 128

  @functools.partial(
      pl.kernel,
      mesh=mesh,
      out_type=jax.ShapeDtypeStruct((D, B), jnp.float32),
      scratch_types=[
          pltpu.VMEM((b_per_w + 16,), jnp.int32),
          [pltpu.VMEM((D, 128), jnp.float32) for _ in range(_RING)],
          pltpu.VMEM((D, b_per_w), jnp.float32),
          pltpu.SemaphoreType.DMA((_RING,)),
      ],
      compiler_params=pltpu.CompilerParams(needs_layout_passes=False),
  )
  def gather_kernel(idx_hbm, table_hbm, aux_hbm, out_hbm, idx_v, slabs, out_v,
                    sems):
    wid = lax.axis_index("s") * info.num_cores + lax.axis_index("c")
    base = wid * b_per_w
    pltpu.sync_copy(idx_hbm.at[pl.ds(base, b_per_w)], idx_v.at[pl.ds(0, b_per_w)])

    rows = [lax.iota(jnp.int32, 16) + 16 * k for k in range(4)]

    def fire(slot, a):
      c = pl.multiple_of((jnp.minimum(a, tail_cut - 1) // 128) * 128, 128)

      @pl.when(a < tail_cut)
      def _():
        pltpu.async_copy(table_hbm.at[:, pl.ds(c, 128)], slabs[slot],
                         sems.at[slot])

      @pl.when(a >= tail_cut)
      def _():
        pltpu.async_copy(aux_hbm, slabs[slot], sems.at[slot])

    vec0 = idx_v[pl.ds(0, 16)]
    for j in range(_RING):
      fire(j, vec0[j])

    def group(i):
      cur = idx_v[pl.ds(i * _RING, 16)]
      nxt = idx_v[pl.ds(i * _RING + _RING, 16)]
      lanes = jnp.where(cur >= tail_cut, cur - tail_start, cur % 128)
      for j in range(_RING):
        a = cur[j]
        lane = jnp.full((16,), lanes[j], jnp.int32)
        pos = jnp.full((16,), i * _RING + j, jnp.int32)
        pltpu.make_async_copy(
            table_hbm.at[:, pl.ds(0, 128)], slabs[j], sems.at[j]
        ).wait()
        for k in range(4):
          vals = plsc.load_gather(slabs[j], [rows[k], lane])
          plsc.store_scatter(out_v, [rows[k], pos], vals)

        @pl.when(i < n_groups - 1)
        def _():
          fire(j, nxt[j])

    pl.loop(0, n_groups)(group)
    pltpu.sync_copy(out_v, out_hbm.at[:, pl.ds(base, b_per_w)])

  return gather_kernel


def kernel(input, action_embedding):
  B = input.shape[0]
  V, D = action_embedding.shape
  idx = input.astype(jnp.int32).reshape(B)
  aux = action_embedding[V - 128:].T
  out_t = _make_gather(V, D, B)(idx, action_embedding.T, aux)
  return out_t.T
